# baseline (speedup 1.0000x reference)
import functools

import jax
import jax.numpy as jnp
from jax import lax
from jax.experimental import pallas as pl
from jax.experimental.pallas import tpu as pltpu, tpu_sc as plsc

P = 32768
A = 2048
ANCHOR_W = 4.0
ANCHOR_L = 4.0
ANCHOR_H = 15.0
NMS_IOU = 0.3
TOPK = 256

NT = 16
PPC = P // NT
APT = A // 32
G = 25
NB = G * G
CAP = 128
OVSTART = NB * CAP
NROWS = OVSTART + P
NBPAD = 640

BA = 256

_mesh = plsc.VectorSubcoreMesh(core_axis_name="c", subcore_axis_name="s")
_sc_params = pltpu.CompilerParams(needs_layout_passes=False)

_i32 = jnp.int32
_f32 = jnp.float32


def _bin_ids(xv, yv):
    bx = jnp.clip((xv * 0.25).astype(_i32), 0, G - 1)
    by = jnp.clip((yv * 0.25).astype(_i32), 0, G - 1)
    return bx * G + by


def _gv(ref, i):
    return jnp.max(plsc.load_gather(ref, [jnp.full((16,), i, _i32)]))


@functools.partial(
    pl.kernel, mesh=_mesh, compiler_params=_sc_params,
    out_type=jax.ShapeDtypeStruct((A,), _f32),
    scratch_types=[pltpu.VMEM((PPC,), _f32),
                   pltpu.VMEM((PPC,), _f32),
                   pltpu.VMEM((PPC,), _f32),
                   pltpu.VMEM((NBPAD,), _i32),
                   pltpu.VMEM((NT * NBPAD,), _i32),
                   pltpu.VMEM((NBPAD,), _i32),
                   pltpu.VMEM((NBPAD,), _i32),
                   pltpu.VMEM((NT * 16,), _i32),
                   pltpu.VMEM((16,), _i32),
                   pltpu.VMEM((APT,), _f32),
                   pltpu.VMEM((APT,), _f32),
                   pltpu.VMEM((APT,), _i32),
                   pltpu.VMEM((APT,), _i32),
                   pltpu.VMEM((APT,), _i32),
                   pltpu.VMEM((APT,), _i32),
                   pltpu.VMEM((2 * CAP,), _f32),
                   pltpu.VMEM((2 * CAP,), _f32),
                   pltpu.VMEM((2 * CAP,), _f32),
                   pltpu.VMEM((2 * CAP,), _f32),
                   pltpu.VMEM((16,), _f32),
                   pltpu.VMEM((16,), _f32),
                   pltpu.VMEM((APT,), _f32),
                   pltpu.VMEM_SHARED((NT * NBPAD,), _i32),
                   pltpu.VMEM_SHARED((NT * 16,), _i32),
                   pltpu.VMEM_SHARED((NROWS,), _f32),
                   pltpu.VMEM_SHARED((NROWS,), _f32),
                   pltpu.SemaphoreType.DMA],
)
def _sc_all(px, py, pz, lmx, lmy, scores, pxl, pyl, pzl, histl, histv, cur,
            tot, ovcv, ovv, lmxl, lmyl, bx0a, bx1a, by0a, by1a,
            bufx0, bufy0, bufx1, bufy1, sacc, cacc, scl,
            hist_sh, ovc_sh, shx, shy, sem):
    cid = lax.axis_index("c")
    sid = lax.axis_index("s")
    wid = cid * NT + sid
    l = lax.iota(_i32, 16)

    base = pl.multiple_of(sid * PPC, PPC)
    pltpu.sync_copy(px.at[pl.ds(base, PPC)], pxl)
    pltpu.sync_copy(py.at[pl.ds(base, PPC)], pyl)
    pltpu.sync_copy(pz.at[pl.ds(base, PPC)], pzl)

    def zero(c, carry):
        histl[pl.ds(pl.multiple_of(c * 16, 16), 16)] = jnp.zeros((16,), _i32)
        return carry

    lax.fori_loop(0, NBPAD // 16, zero, 0)

    def hchunk(k, carry):
        o = pl.multiple_of(k * 16, 16)
        b = _bin_ids(pxl[pl.ds(o, 16)], pyl[pl.ds(o, 16)])
        cnt, last = plsc.scan_count(b)
        c0 = plsc.load_gather(histl, [b])
        plsc.store_scatter(histl, [b], c0 + cnt, mask=last)
        return carry

    lax.fori_loop(0, PPC // 16, hchunk, 0)
    hbase = pl.multiple_of(sid * NBPAD, NBPAD)
    pltpu.sync_copy(histl, hist_sh.at[pl.ds(hbase, NBPAD)])
    plsc.subcore_barrier()

    pltpu.sync_copy(hist_sh, histv)
    sidv = jnp.full((16,), sid, _i32)

    def cursor_chunk(c, carry):
        o = pl.multiple_of(c * 16, 16)
        acc = jnp.zeros((16,), _i32)
        mine = jnp.zeros((16,), _i32)
        for t in range(NT):
            h = histv[pl.ds(t * NBPAD + o, 16)]
            acc += h
            mine += jnp.where(jnp.full((16,), t, _i32) < sidv, h, 0)
        tot[pl.ds(o, 16)] = acc
        cur[pl.ds(o, 16)] = mine
        return carry

    lax.fori_loop(0, NBPAD // 16, cursor_chunk, 0)

    ovbase = OVSTART + sid * PPC

    def pchunk(k, ovcur):
        o = pl.multiple_of(k * 16, 16)
        xv = pxl[pl.ds(o, 16)]
        b = _bin_ids(xv, pyl[pl.ds(o, 16)])
        cnt, last = plsc.scan_count(b)
        c0 = plsc.load_gather(cur, [b])
        slot = c0 + cnt - 1
        plsc.store_scatter(cur, [b], c0 + cnt, mask=last)
        ov = slot >= CAP
        ovr = plsc.cumsum(jnp.where(ov, 1, 0).astype(_i32)) - 1
        dest = jnp.where(ov, ovbase + ovcur + ovr, b * CAP + slot)
        zv = pzl[pl.ds(o, 16)]
        zok = (zv >= 0.0) & (zv <= jnp.float32(ANCHOR_H))
        pzl[pl.ds(o, 16)] = jnp.where(zok, xv, jnp.float32(1e9))
        d1 = pltpu.async_copy(pzl.at[pl.ds(o, 16)], shx.at[dest], sem)
        d2 = pltpu.async_copy(pyl.at[pl.ds(o, 16)], shy.at[dest], sem)
        d1.wait()
        d2.wait()
        return ovcur + jnp.sum(jnp.where(ov, 1, 0).astype(_i32))

    ovcur = lax.fori_loop(0, PPC // 16, pchunk, jnp.int32(0))
    ovv[...] = jnp.full((16,), ovcur, _i32)
    obase = pl.multiple_of(sid * 16, 16)
    pltpu.sync_copy(ovv, ovc_sh.at[pl.ds(obase, 16)])

    abase = pl.multiple_of(wid * APT, APT)
    pltpu.sync_copy(lmx.at[pl.ds(abase, APT)], lmxl)
    pltpu.sync_copy(lmy.at[pl.ds(abase, APT)], lmyl)
    half = jnp.float32(ANCHOR_W / 2)
    for j in range(APT // 16):
        cxv = lmxl[pl.ds(j * 16, 16)]
        cyv = lmyl[pl.ds(j * 16, 16)]
        bx0a[pl.ds(j * 16, 16)] = jnp.clip(((cxv - half) * 0.25).astype(_i32), 0, G - 1)
        bx1a[pl.ds(j * 16, 16)] = jnp.clip(((cxv + half) * 0.25).astype(_i32), 0, G - 1)
        by0a[pl.ds(j * 16, 16)] = jnp.clip(((cyv - half) * 0.25).astype(_i32), 0, G - 1)
        by1a[pl.ds(j * 16, 16)] = jnp.clip(((cyv + half) * 0.25).astype(_i32), 0, G - 1)

    plsc.subcore_barrier()

    pltpu.sync_copy(ovc_sh, ovcv)
    oacc = jnp.zeros((16,), _i32)
    for t in range(NT):
        oacc += ovcv[pl.ds(t * 16, 16)]
    ovtot = jnp.max(oacc)

    hw = jnp.float32(ANCHOR_W / 2)
    denom = hw * hw + jnp.float32(1e-6)

    def anchor_body(a, carry):
        cx = _gv(lmxl, a)
        cy = _gv(lmyl, a)
        bx0 = _gv(bx0a, a)
        bx1 = _gv(bx1a, a)
        by0 = _gv(by0a, a)
        by1 = _gv(by1a, a)
        sacc[...] = jnp.zeros((16,), _f32)
        cacc[...] = jnp.zeros((16,), _f32)
        cxv = jnp.full((16,), cx, _f32)
        cyv = jnp.full((16,), cy, _f32)

        b0 = bx0 * G + by0
        bb0 = pl.multiple_of(b0 * CAP, CAP)
        d0 = pltpu.async_copy(shx.at[pl.ds(bb0, 2 * CAP)], bufx0, sem)
        d1 = pltpu.async_copy(shy.at[pl.ds(bb0, 2 * CAP)], bufy0, sem)
        two_rows = bx1 > bx0

        @pl.when(two_rows)
        def _():
            b1 = bx1 * G + by0
            bb1 = pl.multiple_of(b1 * CAP, CAP)
            d3 = pltpu.async_copy(shx.at[pl.ds(bb1, 2 * CAP)], bufx1, sem)
            d4 = pltpu.async_copy(shy.at[pl.ds(bb1, 2 * CAP)], bufy1, sem)
            d3.wait()
            d4.wait()

        d0.wait()
        d1.wait()

        def accum_chunk(xv, yv, lane_ok):
            inbox = (lane_ok
                     & (xv >= cxv - half) & (xv <= cxv + half)
                     & (yv >= cyv - half) & (yv <= cyv + half))
            dx = xv - cxv
            dy = yv - cyv
            r2 = dx * dx + dy * dy
            w = jnp.exp(-r2 / denom)
            sacc[...] += jnp.where(inbox, w, jnp.float32(0.0))
            cacc[...] += jnp.where(inbox, jnp.float32(1.0), jnp.float32(0.0))

        def row_accum(bx, bufx, bufy):
            def by_body(by, carry3):
                b = bx * G + by
                n = jnp.minimum(_gv(tot, b), CAP)
                off = (by - by0) * CAP

                def chunk(k, carry4):
                    o = pl.multiple_of(off + k * 16, 16)
                    lane_ok = (l + k * 16) < n
                    accum_chunk(bufx[pl.ds(o, 16)], bufy[pl.ds(o, 16)],
                                lane_ok)
                    return carry4

                lax.fori_loop(0, (n + 15) // 16, chunk, 0)
                return carry3

            lax.fori_loop(by0, by1 + 1, by_body, 0)

        row_accum(bx0, bufx0, bufy0)

        @pl.when(two_rows)
        def _():
            row_accum(bx1, bufx1, bufy1)

        @pl.when(ovtot > 0)
        def _():
            def t_body(t, carry2):
                ovt = _gv(ovcv, t * 16)

                def ovchunk(k, carry3):
                    ob = pl.multiple_of(OVSTART + t * PPC + k * 16, 16)
                    pltpu.sync_copy(shx.at[pl.ds(ob, 16)],
                                    bufx0.at[pl.ds(0, 16)])
                    pltpu.sync_copy(shy.at[pl.ds(ob, 16)],
                                    bufy0.at[pl.ds(0, 16)])
                    lane_ok = (l + k * 16) < ovt
                    accum_chunk(bufx0[pl.ds(0, 16)], bufy0[pl.ds(0, 16)],
                                lane_ok)
                    return carry3

                lax.fori_loop(0, (ovt + 15) // 16, ovchunk, 0)
                return carry2

            lax.fori_loop(0, NT, t_body, 0)

        s = jnp.sum(sacc[...])
        c = jnp.sum(cacc[...])
        val = jnp.full((16,), s, _f32) / (jnp.full((16,), c, _f32) + 1.0)
        plsc.store_scatter(scl, [jnp.full((16,), a, _i32)], val, mask=l == 0)
        return carry

    lax.fori_loop(0, APT, anchor_body, 0)
    pltpu.sync_copy(scl, scores.at[pl.ds(abase, APT)])


@functools.partial(
    pl.kernel, mesh=_mesh, compiler_params=_sc_params,
    out_type=jax.ShapeDtypeStruct((A,), _f32),
    scratch_types=[pltpu.VMEM((A,), _f32),
                   pltpu.VMEM((A,), _f32),
                   pltpu.VMEM((A,), _f32),
                   pltpu.VMEM((NBPAD,), _i32),
                   pltpu.VMEM((NBPAD,), _i32),
                   pltpu.VMEM((NBPAD,), _i32),
                   pltpu.VMEM((A,), _i32),
                   pltpu.VMEM((APT,), _f32)],
)
def _sc_nms(lmx, lmy, scores, supp_out, acx, acy, asc, histl, off, cur,
            csr, supl):
    cid = lax.axis_index("c")
    sid = lax.axis_index("s")
    wid = cid * NT + sid
    l = lax.iota(_i32, 16)

    pltpu.sync_copy(lmx, acx)
    pltpu.sync_copy(lmy, acy)
    pltpu.sync_copy(scores, asc)

    def zero(c, carry):
        histl[pl.ds(pl.multiple_of(c * 16, 16), 16)] = jnp.zeros((16,), _i32)
        return carry

    lax.fori_loop(0, NBPAD // 16, zero, 0)

    def hchunk(k, carry):
        o = pl.multiple_of(k * 16, 16)
        b = _bin_ids(acx[pl.ds(o, 16)], acy[pl.ds(o, 16)])
        cnt, last = plsc.scan_count(b)
        c0 = plsc.load_gather(histl, [b])
        plsc.store_scatter(histl, [b], c0 + cnt, mask=last)
        return carry

    lax.fori_loop(0, A // 16, hchunk, 0)

    def offchunk(c, carry):
        o = pl.multiple_of(c * 16, 16)
        t16 = histl[pl.ds(o, 16)]
        cs = plsc.cumsum(t16)
        ex = carry + (cs - t16)
        off[pl.ds(o, 16)] = ex
        cur[pl.ds(o, 16)] = ex
        return carry + jnp.max(cs)

    lax.fori_loop(0, NBPAD // 16, offchunk, jnp.int32(0))

    def cchunk(k, carry):
        o = pl.multiple_of(k * 16, 16)
        b = _bin_ids(acx[pl.ds(o, 16)], acy[pl.ds(o, 16)])
        cnt, last = plsc.scan_count(b)
        c0 = plsc.load_gather(cur, [b])
        plsc.store_scatter(cur, [b], c0 + cnt, mask=last)
        plsc.store_scatter(csr, [c0 + cnt - 1], o + l)
        return carry

    lax.fori_loop(0, A // 16, cchunk, 0)

    halfw = jnp.float32(ANCHOR_W)

    def anchor_body(a, carry):
        ga = wid * APT + a
        gav = jnp.full((16,), ga, _i32)
        cx = _gv(acx, ga)
        cy = _gv(acy, ga)
        sc_i = _gv(asc, ga)
        bx0 = jnp.clip(((cx - halfw) * 0.25).astype(_i32), 0, G - 1)
        bx1 = jnp.clip(((cx + halfw) * 0.25).astype(_i32), 0, G - 1)
        by0 = jnp.clip(((cy - halfw) * 0.25).astype(_i32), 0, G - 1)
        by1 = jnp.clip(((cy + halfw) * 0.25).astype(_i32), 0, G - 1)

        half = jnp.float32(ANCHOR_W / 2)
        x1i = jnp.full((16,), cx - half, _f32)
        x2i = jnp.full((16,), cx + half, _f32)
        y1i = jnp.full((16,), cy - half, _f32)
        y2i = jnp.full((16,), cy + half, _f32)
        area_i = (x2i - x1i) * (y2i - y1i)
        sciv = jnp.full((16,), sc_i, _f32)

        def bx_body(bx, acc):
            bfirst = bx * G + by0
            blast = bx * G + by1
            start = _gv(off, bfirst)
            end = _gv(off, blast) + _gv(histl, blast)

            def chunk(k, acc2):
                ix = start + k * 16 + l
                lane_ok = ix < end
                aj = plsc.load_gather(csr, [jnp.minimum(ix, A - 1)])
                cxj = plsc.load_gather(acx, [aj])
                cyj = plsc.load_gather(acy, [aj])
                scj = plsc.load_gather(asc, [aj])
                x1j = cxj - half
                x2j = cxj + half
                y1j = cyj - half
                y2j = cyj + half
                area_j = (x2j - x1j) * (y2j - y1j)
                iw = jnp.maximum(jnp.minimum(x2i, x2j) - jnp.maximum(x1i, x1j),
                                 jnp.float32(0.0))
                ih = jnp.maximum(jnp.minimum(y2i, y2j) - jnp.maximum(y1i, y1j),
                                 jnp.float32(0.0))
                inter = iw * ih
                union = area_i + area_j - inter
                iou = inter / (union + jnp.float32(1e-9))
                higher = (scj > sciv) | ((scj == sciv) & (aj < gav))
                cond = lane_ok & higher & (iou > jnp.float32(NMS_IOU))
                return acc2 | jnp.any(cond)

            return lax.fori_loop(0, (end - start + 15) // 16, chunk, acc)

        suppressed = lax.fori_loop(bx0, bx1 + 1, bx_body, jnp.bool_(False))
        val = jnp.where(suppressed, jnp.float32(1.0), jnp.float32(0.0))
        plsc.store_scatter(supl, [jnp.full((16,), a, _i32)],
                           jnp.full((16,), val, _f32), mask=l == 0)
        return carry

    lax.fori_loop(0, APT, anchor_body, 0)
    abase = pl.multiple_of(wid * APT, APT)
    pltpu.sync_copy(supl, supp_out.at[pl.ds(abase, APT)])


def _nms_body(score_c, score_r, cx_c, cy_c, cx_r, cy_r, supp_ref):
    s_i = score_c[...]
    s_j = score_r[...]

    half_w = jnp.float32(ANCHOR_W) / 2
    half_l = jnp.float32(ANCHOR_L) / 2
    x1_i = cx_c[...] - half_w
    y1_i = cy_c[...] - half_l
    x2_i = cx_c[...] + half_w
    y2_i = cy_c[...] + half_l
    x1_j = cx_r[...] - half_w
    y1_j = cy_r[...] - half_l
    x2_j = cx_r[...] + half_w
    y2_j = cy_r[...] + half_l
    area_i = (x2_i - x1_i) * (y2_i - y1_i)
    area_j = (x2_j - x1_j) * (y2_j - y1_j)
    ix1 = jnp.maximum(x1_i, x1_j)
    iy1 = jnp.maximum(y1_i, y1_j)
    ix2 = jnp.minimum(x2_i, x2_j)
    iy2 = jnp.minimum(y2_i, y2_j)
    iw = jnp.clip(ix2 - ix1, 0.0, None)
    ih = jnp.clip(iy2 - iy1, 0.0, None)
    inter = iw * ih
    union = area_i + area_j - inter
    iou = inter / (union + jnp.float32(1e-9))

    i_blk = pl.program_id(0)
    idx_i = i_blk * BA + jax.lax.broadcasted_iota(_i32, (BA, A), 0)
    idx_j = jax.lax.broadcasted_iota(_i32, (BA, A), 1)
    higher = (s_j > s_i) | ((s_j == s_i) & (idx_j < idx_i))
    suppressed = jnp.any(higher & (iou > jnp.float32(NMS_IOU)), axis=1,
                         keepdims=True)
    supp_ref[...] = suppressed.astype(_f32)


def _topk_body(score_c, supp_c, score_r, supp_r, cx_r, cy_r,
               boxes_ref, top_ref):
    neg_inf = jnp.float32(-jnp.inf)
    m_r = jnp.where(supp_r[...] > 0, neg_inf, score_r[...])
    rank = jnp.zeros((1, A), _i32)
    idx_j = jax.lax.broadcasted_iota(_i32, (BA, A), 1)
    for ib in range(A // BA):
        m_i = jnp.where(supp_c[pl.ds(ib * BA, BA), :] > 0, neg_inf,
                        score_c[pl.ds(ib * BA, BA), :])
        idx_i = ib * BA + jax.lax.broadcasted_iota(_i32, (BA, A), 0)
        ahead = (m_i > m_r) | ((m_i == m_r) & (idx_i < idx_j))
        rank = rank + ahead.astype(_i32).sum(axis=0, keepdims=True)

    m_j = m_r
    k = jax.lax.broadcasted_iota(_i32, (TOPK, A), 0)
    eq = rank == k
    zero = jnp.float32(0.0)
    top_ref[...] = jnp.where(eq, m_j, zero).sum(axis=1, keepdims=True)
    bx = jnp.where(eq, cx_r[...], zero).sum(axis=1, keepdims=True)
    by = jnp.where(eq, cy_r[...], zero).sum(axis=1, keepdims=True)
    ones = jnp.ones((TOPK, 1), _f32)
    boxes_ref[...] = jnp.concatenate(
        [bx, by, jnp.zeros((TOPK, 1), _f32),
         ones * jnp.float32(ANCHOR_W), ones * jnp.float32(ANCHOR_L),
         ones * jnp.float32(ANCHOR_H)], axis=1)


def kernel(points, gt_boxes, local_maxima, plot_bounds, training):
    del gt_boxes, plot_bounds, training
    px = points[:, 0].astype(_f32)
    py = points[:, 1].astype(_f32)
    pz = points[:, 2].astype(_f32)
    lmx = local_maxima[:, 0].astype(_f32)
    lmy = local_maxima[:, 1].astype(_f32)

    score_flat = _sc_all(px, py, pz, lmx, lmy)
    supp_flat = _sc_nms(lmx, lmy, score_flat)

    score = score_flat.reshape(A, 1)
    score_r = score_flat.reshape(1, A)
    supp = supp_flat.reshape(A, 1)
    supp_r = supp_flat.reshape(1, A)
    cx_r = lmx.reshape(1, A)
    cy_r = lmy.reshape(1, A)

    fullc = pl.BlockSpec((A, 1), lambda: (0, 0))
    fullr = pl.BlockSpec((1, A), lambda: (0, 0))
    boxes, top = pl.pallas_call(
        _topk_body,
        in_specs=[fullc, fullc, fullr, fullr, fullr, fullr],
        out_specs=[pl.BlockSpec((TOPK, 6), lambda: (0, 0)),
                   pl.BlockSpec((TOPK, 1), lambda: (0, 0))],
        out_shape=[jax.ShapeDtypeStruct((TOPK, 6), _f32),
                   jax.ShapeDtypeStruct((TOPK, 1), _f32)],
    )(score, supp, score_r, supp_r, cx_r, cy_r)

    return boxes, top.reshape(TOPK)


# speedup vs baseline: 1.1073x; 1.1073x over previous
import functools

import jax
import jax.numpy as jnp
from jax import lax
from jax.experimental import pallas as pl
from jax.experimental.pallas import tpu as pltpu, tpu_sc as plsc

P = 32768
A = 2048
ANCHOR_W = 4.0
ANCHOR_L = 4.0
ANCHOR_H = 15.0
NMS_IOU = 0.3
TOPK = 256

NT = 16
PPC = P // NT
APT = A // 32
G = 25
NB = G * G
CAP = 128
OVSTART = NB * CAP
NROWS = OVSTART + P
NBPAD = 640

BA = 256

_mesh = plsc.VectorSubcoreMesh(core_axis_name="c", subcore_axis_name="s")
_sc_params = pltpu.CompilerParams(needs_layout_passes=False)

_i32 = jnp.int32
_f32 = jnp.float32


def _bin_ids(xv, yv):
    bx = jnp.clip((xv * 0.25).astype(_i32), 0, G - 1)
    by = jnp.clip((yv * 0.25).astype(_i32), 0, G - 1)
    return bx * G + by


def _gv(ref, i):
    return jnp.max(plsc.load_gather(ref, [jnp.full((16,), i, _i32)]))


@functools.partial(
    pl.kernel, mesh=_mesh, compiler_params=_sc_params,
    out_type=jax.ShapeDtypeStruct((A,), _f32),
    scratch_types=[pltpu.VMEM((PPC,), _f32),
                   pltpu.VMEM((PPC,), _f32),
                   pltpu.VMEM((PPC,), _f32),
                   pltpu.VMEM((NBPAD,), _i32),
                   pltpu.VMEM((NT * NBPAD,), _i32),
                   pltpu.VMEM((NBPAD,), _i32),
                   pltpu.VMEM((NBPAD,), _i32),
                   pltpu.VMEM((NT * 16,), _i32),
                   pltpu.VMEM((16,), _i32),
                   pltpu.VMEM((APT,), _f32),
                   pltpu.VMEM((APT,), _f32),
                   pltpu.VMEM((APT,), _i32),
                   pltpu.VMEM((APT,), _i32),
                   pltpu.VMEM((APT,), _i32),
                   pltpu.VMEM((APT,), _i32),
                   pltpu.VMEM((2 * CAP,), _f32),
                   pltpu.VMEM((2 * CAP,), _f32),
                   pltpu.VMEM((2 * CAP,), _f32),
                   pltpu.VMEM((2 * CAP,), _f32),
                   pltpu.VMEM((16,), _f32),
                   pltpu.VMEM((16,), _f32),
                   pltpu.VMEM((APT,), _f32),
                   pltpu.VMEM_SHARED((NT * NBPAD,), _i32),
                   pltpu.VMEM_SHARED((NT * 16,), _i32),
                   pltpu.VMEM_SHARED((NROWS,), _f32),
                   pltpu.VMEM_SHARED((NROWS,), _f32),
                   pltpu.SemaphoreType.DMA],
)
def _sc_all(px, py, pz, lmx, lmy, scores, pxl, pyl, pzl, histl, histv, cur,
            tot, ovcv, ovv, lmxl, lmyl, bx0a, bx1a, by0a, by1a,
            bufx0, bufy0, bufx1, bufy1, sacc, cacc, scl,
            hist_sh, ovc_sh, shx, shy, sem):
    cid = lax.axis_index("c")
    sid = lax.axis_index("s")
    wid = cid * NT + sid
    l = lax.iota(_i32, 16)

    base = pl.multiple_of(sid * PPC, PPC)
    pltpu.sync_copy(px.at[pl.ds(base, PPC)], pxl)
    pltpu.sync_copy(py.at[pl.ds(base, PPC)], pyl)
    pltpu.sync_copy(pz.at[pl.ds(base, PPC)], pzl)

    def zero(c, carry):
        histl[pl.ds(pl.multiple_of(c * 16, 16), 16)] = jnp.zeros((16,), _i32)
        return carry

    lax.fori_loop(0, NBPAD // 16, zero, 0)

    def hchunk(k, carry):
        o = pl.multiple_of(k * 16, 16)
        b = _bin_ids(pxl[pl.ds(o, 16)], pyl[pl.ds(o, 16)])
        cnt, last = plsc.scan_count(b)
        c0 = plsc.load_gather(histl, [b])
        plsc.store_scatter(histl, [b], c0 + cnt, mask=last)
        return carry

    lax.fori_loop(0, PPC // 16, hchunk, 0)
    hbase = pl.multiple_of(sid * NBPAD, NBPAD)
    pltpu.sync_copy(histl, hist_sh.at[pl.ds(hbase, NBPAD)])
    plsc.subcore_barrier()

    pltpu.sync_copy(hist_sh, histv)
    sidv = jnp.full((16,), sid, _i32)

    def cursor_chunk(c, carry):
        o = pl.multiple_of(c * 16, 16)
        acc = jnp.zeros((16,), _i32)
        mine = jnp.zeros((16,), _i32)
        for t in range(NT):
            h = histv[pl.ds(t * NBPAD + o, 16)]
            acc += h
            mine += jnp.where(jnp.full((16,), t, _i32) < sidv, h, 0)
        tot[pl.ds(o, 16)] = acc
        cur[pl.ds(o, 16)] = mine
        return carry

    lax.fori_loop(0, NBPAD // 16, cursor_chunk, 0)

    ovbase = OVSTART + sid * PPC

    def pchunk(k, ovcur):
        o = pl.multiple_of(k * 16, 16)
        xv = pxl[pl.ds(o, 16)]
        b = _bin_ids(xv, pyl[pl.ds(o, 16)])
        cnt, last = plsc.scan_count(b)
        c0 = plsc.load_gather(cur, [b])
        slot = c0 + cnt - 1
        plsc.store_scatter(cur, [b], c0 + cnt, mask=last)
        ov = slot >= CAP
        ovr = plsc.cumsum(jnp.where(ov, 1, 0).astype(_i32)) - 1
        dest = jnp.where(ov, ovbase + ovcur + ovr, b * CAP + slot)
        zv = pzl[pl.ds(o, 16)]
        zok = (zv >= 0.0) & (zv <= jnp.float32(ANCHOR_H))
        pzl[pl.ds(o, 16)] = jnp.where(zok, xv, jnp.float32(1e9))
        d1 = pltpu.async_copy(pzl.at[pl.ds(o, 16)], shx.at[dest], sem)
        d2 = pltpu.async_copy(pyl.at[pl.ds(o, 16)], shy.at[dest], sem)
        d1.wait()
        d2.wait()
        return ovcur + jnp.sum(jnp.where(ov, 1, 0).astype(_i32))

    ovcur = lax.fori_loop(0, PPC // 16, pchunk, jnp.int32(0))
    ovv[...] = jnp.full((16,), ovcur, _i32)
    obase = pl.multiple_of(sid * 16, 16)
    pltpu.sync_copy(ovv, ovc_sh.at[pl.ds(obase, 16)])

    abase = pl.multiple_of(wid * APT, APT)
    pltpu.sync_copy(lmx.at[pl.ds(abase, APT)], lmxl)
    pltpu.sync_copy(lmy.at[pl.ds(abase, APT)], lmyl)
    half = jnp.float32(ANCHOR_W / 2)
    for j in range(APT // 16):
        cxv = lmxl[pl.ds(j * 16, 16)]
        cyv = lmyl[pl.ds(j * 16, 16)]
        bx0a[pl.ds(j * 16, 16)] = jnp.clip(((cxv - half) * 0.25).astype(_i32), 0, G - 1)
        bx1a[pl.ds(j * 16, 16)] = jnp.clip(((cxv + half) * 0.25).astype(_i32), 0, G - 1)
        by0a[pl.ds(j * 16, 16)] = jnp.clip(((cyv - half) * 0.25).astype(_i32), 0, G - 1)
        by1a[pl.ds(j * 16, 16)] = jnp.clip(((cyv + half) * 0.25).astype(_i32), 0, G - 1)

    plsc.subcore_barrier()

    pltpu.sync_copy(ovc_sh, ovcv)
    oacc = jnp.zeros((16,), _i32)
    for t in range(NT):
        oacc += ovcv[pl.ds(t * 16, 16)]
    ovtot = jnp.max(oacc)

    hw = jnp.float32(ANCHOR_W / 2)
    denom = hw * hw + jnp.float32(1e-6)

    def anchor_body(a, carry):
        cx = _gv(lmxl, a)
        cy = _gv(lmyl, a)
        bx0 = _gv(bx0a, a)
        bx1 = _gv(bx1a, a)
        by0 = _gv(by0a, a)
        by1 = _gv(by1a, a)
        sacc[...] = jnp.zeros((16,), _f32)
        cacc[...] = jnp.zeros((16,), _f32)
        cxv = jnp.full((16,), cx, _f32)
        cyv = jnp.full((16,), cy, _f32)

        b0 = bx0 * G + by0
        bb0 = pl.multiple_of(b0 * CAP, CAP)
        d0 = pltpu.async_copy(shx.at[pl.ds(bb0, 2 * CAP)], bufx0, sem)
        d1 = pltpu.async_copy(shy.at[pl.ds(bb0, 2 * CAP)], bufy0, sem)
        two_rows = bx1 > bx0

        @pl.when(two_rows)
        def _():
            b1 = bx1 * G + by0
            bb1 = pl.multiple_of(b1 * CAP, CAP)
            d3 = pltpu.async_copy(shx.at[pl.ds(bb1, 2 * CAP)], bufx1, sem)
            d4 = pltpu.async_copy(shy.at[pl.ds(bb1, 2 * CAP)], bufy1, sem)
            d3.wait()
            d4.wait()

        d0.wait()
        d1.wait()

        def accum_chunk(xv, yv, lane_ok):
            inbox = (lane_ok
                     & (xv >= cxv - half) & (xv <= cxv + half)
                     & (yv >= cyv - half) & (yv <= cyv + half))
            dx = xv - cxv
            dy = yv - cyv
            r2 = dx * dx + dy * dy
            w = jnp.exp(-r2 / denom)
            sacc[...] += jnp.where(inbox, w, jnp.float32(0.0))
            cacc[...] += jnp.where(inbox, jnp.float32(1.0), jnp.float32(0.0))

        def row_accum(bx, bufx, bufy):
            def by_body(by, carry3):
                b = bx * G + by
                n = jnp.minimum(_gv(tot, b), CAP)
                off = (by - by0) * CAP

                def chunk(k, carry4):
                    o = pl.multiple_of(off + k * 16, 16)
                    lane_ok = (l + k * 16) < n
                    accum_chunk(bufx[pl.ds(o, 16)], bufy[pl.ds(o, 16)],
                                lane_ok)
                    return carry4

                lax.fori_loop(0, (n + 15) // 16, chunk, 0)
                return carry3

            lax.fori_loop(by0, by1 + 1, by_body, 0)

        row_accum(bx0, bufx0, bufy0)

        @pl.when(two_rows)
        def _():
            row_accum(bx1, bufx1, bufy1)

        @pl.when(ovtot > 0)
        def _():
            def t_body(t, carry2):
                ovt = _gv(ovcv, t * 16)

                def ovchunk(k, carry3):
                    ob = pl.multiple_of(OVSTART + t * PPC + k * 16, 16)
                    pltpu.sync_copy(shx.at[pl.ds(ob, 16)],
                                    bufx0.at[pl.ds(0, 16)])
                    pltpu.sync_copy(shy.at[pl.ds(ob, 16)],
                                    bufy0.at[pl.ds(0, 16)])
                    lane_ok = (l + k * 16) < ovt
                    accum_chunk(bufx0[pl.ds(0, 16)], bufy0[pl.ds(0, 16)],
                                lane_ok)
                    return carry3

                lax.fori_loop(0, (ovt + 15) // 16, ovchunk, 0)
                return carry2

            lax.fori_loop(0, NT, t_body, 0)

        s = jnp.sum(sacc[...])
        c = jnp.sum(cacc[...])
        val = jnp.full((16,), s, _f32) / (jnp.full((16,), c, _f32) + 1.0)
        plsc.store_scatter(scl, [jnp.full((16,), a, _i32)], val, mask=l == 0)
        return carry

    lax.fori_loop(0, APT, anchor_body, 0)
    pltpu.sync_copy(scl, scores.at[pl.ds(abase, APT)])


@functools.partial(
    pl.kernel, mesh=_mesh, compiler_params=_sc_params,
    out_type=jax.ShapeDtypeStruct((A,), _f32),
    scratch_types=[pltpu.VMEM((A,), _f32),
                   pltpu.VMEM((A,), _f32),
                   pltpu.VMEM((A,), _f32),
                   pltpu.VMEM((NBPAD,), _i32),
                   pltpu.VMEM((NBPAD,), _i32),
                   pltpu.VMEM((NBPAD,), _i32),
                   pltpu.VMEM((A,), _i32),
                   pltpu.VMEM((APT,), _f32)],
)
def _sc_nms(lmx, lmy, scores, supp_out, acx, acy, asc, histl, off, cur,
            csr, supl):
    cid = lax.axis_index("c")
    sid = lax.axis_index("s")
    wid = cid * NT + sid
    l = lax.iota(_i32, 16)

    pltpu.sync_copy(lmx, acx)
    pltpu.sync_copy(lmy, acy)
    pltpu.sync_copy(scores, asc)

    def zero(c, carry):
        histl[pl.ds(pl.multiple_of(c * 16, 16), 16)] = jnp.zeros((16,), _i32)
        return carry

    lax.fori_loop(0, NBPAD // 16, zero, 0)

    def hchunk(k, carry):
        o = pl.multiple_of(k * 16, 16)
        b = _bin_ids(acx[pl.ds(o, 16)], acy[pl.ds(o, 16)])
        cnt, last = plsc.scan_count(b)
        c0 = plsc.load_gather(histl, [b])
        plsc.store_scatter(histl, [b], c0 + cnt, mask=last)
        return carry

    lax.fori_loop(0, A // 16, hchunk, 0)

    def offchunk(c, carry):
        o = pl.multiple_of(c * 16, 16)
        t16 = histl[pl.ds(o, 16)]
        cs = plsc.cumsum(t16)
        ex = carry + (cs - t16)
        off[pl.ds(o, 16)] = ex
        cur[pl.ds(o, 16)] = ex
        return carry + jnp.max(cs)

    lax.fori_loop(0, NBPAD // 16, offchunk, jnp.int32(0))

    def cchunk(k, carry):
        o = pl.multiple_of(k * 16, 16)
        b = _bin_ids(acx[pl.ds(o, 16)], acy[pl.ds(o, 16)])
        cnt, last = plsc.scan_count(b)
        c0 = plsc.load_gather(cur, [b])
        plsc.store_scatter(cur, [b], c0 + cnt, mask=last)
        plsc.store_scatter(csr, [c0 + cnt - 1], o + l)
        return carry

    lax.fori_loop(0, A // 16, cchunk, 0)

    abase = pl.multiple_of(wid * APT, APT)
    half = jnp.float32(ANCHOR_W / 2)

    def chunk_body(j, carry):
        o = pl.multiple_of(abase + j * 16, 16)
        cxv = acx[pl.ds(o, 16)]
        cyv = acy[pl.ds(o, 16)]
        scv = asc[pl.ds(o, 16)]
        gav = o + l
        bxv = jnp.clip((cxv * 0.25).astype(_i32), 0, G - 1)
        byv = jnp.clip((cyv * 0.25).astype(_i32), 0, G - 1)
        x1i = cxv - half
        x2i = cxv + half
        y1i = cyv - half
        y2i = cyv + half
        area_i = (x2i - x1i) * (y2i - y1i)

        supv = jnp.zeros((16,), jnp.bool_)
        for dx in (-1, 0, 1):
            for dy in (-1, 0, 1):
                bn = (jnp.clip(bxv + dx, 0, G - 1) * G
                      + jnp.clip(byv + dy, 0, G - 1))
                offv = plsc.load_gather(off, [bn])
                cntv = plsc.load_gather(histl, [bn])
                mx = jnp.max(cntv)

                def kbody(k, sup):
                    lane_ok = k < cntv
                    aj = plsc.load_gather(
                        csr, [jnp.minimum(offv + k, A - 1)])
                    cxj = plsc.load_gather(acx, [aj])
                    cyj = plsc.load_gather(acy, [aj])
                    scj = plsc.load_gather(asc, [aj])
                    x1j = cxj - half
                    x2j = cxj + half
                    y1j = cyj - half
                    y2j = cyj + half
                    area_j = (x2j - x1j) * (y2j - y1j)
                    iw = jnp.maximum(
                        jnp.minimum(x2i, x2j) - jnp.maximum(x1i, x1j),
                        jnp.float32(0.0))
                    ih = jnp.maximum(
                        jnp.minimum(y2i, y2j) - jnp.maximum(y1i, y1j),
                        jnp.float32(0.0))
                    inter = iw * ih
                    union = area_i + area_j - inter
                    iou = inter / (union + jnp.float32(1e-9))
                    higher = (scj > scv) | ((scj == scv) & (aj < gav))
                    cond = lane_ok & higher & (iou > jnp.float32(NMS_IOU))
                    return sup | cond

                supv = lax.fori_loop(0, mx, kbody, supv)

        supl[pl.ds(pl.multiple_of(j * 16, 16), 16)] = jnp.where(
            supv, jnp.float32(1.0), jnp.float32(0.0))
        return carry

    lax.fori_loop(0, APT // 16, chunk_body, 0)
    pltpu.sync_copy(supl, supp_out.at[pl.ds(abase, APT)])


def _nms_body(score_c, score_r, cx_c, cy_c, cx_r, cy_r, supp_ref):
    s_i = score_c[...]
    s_j = score_r[...]

    half_w = jnp.float32(ANCHOR_W) / 2
    half_l = jnp.float32(ANCHOR_L) / 2
    x1_i = cx_c[...] - half_w
    y1_i = cy_c[...] - half_l
    x2_i = cx_c[...] + half_w
    y2_i = cy_c[...] + half_l
    x1_j = cx_r[...] - half_w
    y1_j = cy_r[...] - half_l
    x2_j = cx_r[...] + half_w
    y2_j = cy_r[...] + half_l
    area_i = (x2_i - x1_i) * (y2_i - y1_i)
    area_j = (x2_j - x1_j) * (y2_j - y1_j)
    ix1 = jnp.maximum(x1_i, x1_j)
    iy1 = jnp.maximum(y1_i, y1_j)
    ix2 = jnp.minimum(x2_i, x2_j)
    iy2 = jnp.minimum(y2_i, y2_j)
    iw = jnp.clip(ix2 - ix1, 0.0, None)
    ih = jnp.clip(iy2 - iy1, 0.0, None)
    inter = iw * ih
    union = area_i + area_j - inter
    iou = inter / (union + jnp.float32(1e-9))

    i_blk = pl.program_id(0)
    idx_i = i_blk * BA + jax.lax.broadcasted_iota(_i32, (BA, A), 0)
    idx_j = jax.lax.broadcasted_iota(_i32, (BA, A), 1)
    higher = (s_j > s_i) | ((s_j == s_i) & (idx_j < idx_i))
    suppressed = jnp.any(higher & (iou > jnp.float32(NMS_IOU)), axis=1,
                         keepdims=True)
    supp_ref[...] = suppressed.astype(_f32)


def _topk_body(score_c, supp_c, score_r, supp_r, cx_r, cy_r,
               boxes_ref, top_ref):
    neg_inf = jnp.float32(-jnp.inf)
    m_r = jnp.where(supp_r[...] > 0, neg_inf, score_r[...])
    rank = jnp.zeros((1, A), _i32)
    idx_j = jax.lax.broadcasted_iota(_i32, (BA, A), 1)
    for ib in range(A // BA):
        m_i = jnp.where(supp_c[pl.ds(ib * BA, BA), :] > 0, neg_inf,
                        score_c[pl.ds(ib * BA, BA), :])
        idx_i = ib * BA + jax.lax.broadcasted_iota(_i32, (BA, A), 0)
        ahead = (m_i > m_r) | ((m_i == m_r) & (idx_i < idx_j))
        rank = rank + ahead.astype(_i32).sum(axis=0, keepdims=True)

    m_j = m_r
    k = jax.lax.broadcasted_iota(_i32, (TOPK, A), 0)
    eq = rank == k
    zero = jnp.float32(0.0)
    top_ref[...] = jnp.where(eq, m_j, zero).sum(axis=1, keepdims=True)
    bx = jnp.where(eq, cx_r[...], zero).sum(axis=1, keepdims=True)
    by = jnp.where(eq, cy_r[...], zero).sum(axis=1, keepdims=True)
    ones = jnp.ones((TOPK, 1), _f32)
    boxes_ref[...] = jnp.concatenate(
        [bx, by, jnp.zeros((TOPK, 1), _f32),
         ones * jnp.float32(ANCHOR_W), ones * jnp.float32(ANCHOR_L),
         ones * jnp.float32(ANCHOR_H)], axis=1)


def kernel(points, gt_boxes, local_maxima, plot_bounds, training):
    del gt_boxes, plot_bounds, training
    px = points[:, 0].astype(_f32)
    py = points[:, 1].astype(_f32)
    pz = points[:, 2].astype(_f32)
    lmx = local_maxima[:, 0].astype(_f32)
    lmy = local_maxima[:, 1].astype(_f32)

    score_flat = _sc_all(px, py, pz, lmx, lmy)
    supp_flat = _sc_nms(lmx, lmy, score_flat)

    score = score_flat.reshape(A, 1)
    score_r = score_flat.reshape(1, A)
    supp = supp_flat.reshape(A, 1)
    supp_r = supp_flat.reshape(1, A)
    cx_r = lmx.reshape(1, A)
    cy_r = lmy.reshape(1, A)

    fullc = pl.BlockSpec((A, 1), lambda: (0, 0))
    fullr = pl.BlockSpec((1, A), lambda: (0, 0))
    boxes, top = pl.pallas_call(
        _topk_body,
        in_specs=[fullc, fullc, fullr, fullr, fullr, fullr],
        out_specs=[pl.BlockSpec((TOPK, 6), lambda: (0, 0)),
                   pl.BlockSpec((TOPK, 1), lambda: (0, 0))],
        out_shape=[jax.ShapeDtypeStruct((TOPK, 6), _f32),
                   jax.ShapeDtypeStruct((TOPK, 1), _f32)],
    )(score, supp, score_r, supp_r, cx_r, cy_r)

    return boxes, top.reshape(TOPK)


# speedup vs baseline: 1.1269x; 1.0176x over previous
import functools

import jax
import jax.numpy as jnp
from jax import lax
from jax.experimental import pallas as pl
from jax.experimental.pallas import tpu as pltpu, tpu_sc as plsc

P = 32768
A = 2048
ANCHOR_W = 4.0
ANCHOR_L = 4.0
ANCHOR_H = 15.0
NMS_IOU = 0.3
TOPK = 256

NT = 16
PPC = P // NT
APT = A // 32
G = 25
NB = G * G
CAP = 96
OVSTART = NB * CAP
NROWS = OVSTART + P
NBPAD = 640

BA = 256

_mesh = plsc.VectorSubcoreMesh(core_axis_name="c", subcore_axis_name="s")
_sc_params = pltpu.CompilerParams(needs_layout_passes=False)

_i32 = jnp.int32
_f32 = jnp.float32


def _bin_ids(xv, yv):
    bx = jnp.clip((xv * 0.25).astype(_i32), 0, G - 1)
    by = jnp.clip((yv * 0.25).astype(_i32), 0, G - 1)
    return bx * G + by


def _gv(ref, i):
    return jnp.max(plsc.load_gather(ref, [jnp.full((16,), i, _i32)]))


@functools.partial(
    pl.kernel, mesh=_mesh, compiler_params=_sc_params,
    out_type=jax.ShapeDtypeStruct((A,), _f32),
    scratch_types=[pltpu.VMEM((PPC,), _f32),
                   pltpu.VMEM((PPC,), _f32),
                   pltpu.VMEM((PPC,), _f32),
                   pltpu.VMEM((NBPAD,), _i32),
                   pltpu.VMEM((NT * NBPAD,), _i32),
                   pltpu.VMEM((NBPAD,), _i32),
                   pltpu.VMEM((NBPAD,), _i32),
                   pltpu.VMEM((NT * 16,), _i32),
                   pltpu.VMEM((16,), _i32),
                   pltpu.VMEM((APT,), _f32),
                   pltpu.VMEM((APT,), _f32),
                   pltpu.VMEM((APT,), _i32),
                   pltpu.VMEM((APT,), _i32),
                   pltpu.VMEM((APT,), _i32),
                   pltpu.VMEM((APT,), _i32),
                   pltpu.VMEM((2 * CAP,), _f32),
                   pltpu.VMEM((2 * CAP,), _f32),
                   pltpu.VMEM((2 * CAP,), _f32),
                   pltpu.VMEM((2 * CAP,), _f32),
                   pltpu.VMEM((16,), _f32),
                   pltpu.VMEM((16,), _f32),
                   pltpu.VMEM((APT,), _f32),
                   pltpu.VMEM_SHARED((NT * NBPAD,), _i32),
                   pltpu.VMEM_SHARED((NT * 16,), _i32),
                   pltpu.VMEM_SHARED((NROWS,), _f32),
                   pltpu.VMEM_SHARED((NROWS,), _f32),
                   pltpu.SemaphoreType.DMA],
)
def _sc_all(px, py, pz, lmx, lmy, scores, pxl, pyl, pzl, histl, histv, cur,
            tot, ovcv, ovv, lmxl, lmyl, bx0a, bx1a, by0a, by1a,
            bufx0, bufy0, bufx1, bufy1, sacc, cacc, scl,
            hist_sh, ovc_sh, shx, shy, sem):
    cid = lax.axis_index("c")
    sid = lax.axis_index("s")
    wid = cid * NT + sid
    l = lax.iota(_i32, 16)

    base = pl.multiple_of(sid * PPC, PPC)
    pltpu.sync_copy(px.at[pl.ds(base, PPC)], pxl)
    pltpu.sync_copy(py.at[pl.ds(base, PPC)], pyl)
    pltpu.sync_copy(pz.at[pl.ds(base, PPC)], pzl)

    def zero(c, carry):
        histl[pl.ds(pl.multiple_of(c * 16, 16), 16)] = jnp.zeros((16,), _i32)
        return carry

    lax.fori_loop(0, NBPAD // 16, zero, 0)

    def hchunk(k, carry):
        o = pl.multiple_of(k * 16, 16)
        b = _bin_ids(pxl[pl.ds(o, 16)], pyl[pl.ds(o, 16)])
        cnt, last = plsc.scan_count(b)
        c0 = plsc.load_gather(histl, [b])
        plsc.store_scatter(histl, [b], c0 + cnt, mask=last)
        return carry

    lax.fori_loop(0, PPC // 16, hchunk, 0)
    hbase = pl.multiple_of(sid * NBPAD, NBPAD)
    pltpu.sync_copy(histl, hist_sh.at[pl.ds(hbase, NBPAD)])
    plsc.subcore_barrier()

    pltpu.sync_copy(hist_sh, histv)
    sidv = jnp.full((16,), sid, _i32)

    def cursor_chunk(c, carry):
        o = pl.multiple_of(c * 16, 16)
        acc = jnp.zeros((16,), _i32)
        mine = jnp.zeros((16,), _i32)
        for t in range(NT):
            h = histv[pl.ds(t * NBPAD + o, 16)]
            acc += h
            mine += jnp.where(jnp.full((16,), t, _i32) < sidv, h, 0)
        tot[pl.ds(o, 16)] = acc
        cur[pl.ds(o, 16)] = mine
        return carry

    lax.fori_loop(0, NBPAD // 16, cursor_chunk, 0)

    ovbase = OVSTART + sid * PPC

    def pchunk(k, ovcur):
        o = pl.multiple_of(k * 16, 16)
        xv = pxl[pl.ds(o, 16)]
        b = _bin_ids(xv, pyl[pl.ds(o, 16)])
        cnt, last = plsc.scan_count(b)
        c0 = plsc.load_gather(cur, [b])
        slot = c0 + cnt - 1
        plsc.store_scatter(cur, [b], c0 + cnt, mask=last)
        ov = slot >= CAP
        ovr = plsc.cumsum(jnp.where(ov, 1, 0).astype(_i32)) - 1
        dest = jnp.where(ov, ovbase + ovcur + ovr, b * CAP + slot)
        zv = pzl[pl.ds(o, 16)]
        zok = (zv >= 0.0) & (zv <= jnp.float32(ANCHOR_H))
        pzl[pl.ds(o, 16)] = jnp.where(zok, xv, jnp.float32(1e9))
        d1 = pltpu.async_copy(pzl.at[pl.ds(o, 16)], shx.at[dest], sem)
        d2 = pltpu.async_copy(pyl.at[pl.ds(o, 16)], shy.at[dest], sem)
        d1.wait()
        d2.wait()
        return ovcur + jnp.sum(jnp.where(ov, 1, 0).astype(_i32))

    ovcur = lax.fori_loop(0, PPC // 16, pchunk, jnp.int32(0))
    ovv[...] = jnp.full((16,), ovcur, _i32)
    obase = pl.multiple_of(sid * 16, 16)
    pltpu.sync_copy(ovv, ovc_sh.at[pl.ds(obase, 16)])

    abase = pl.multiple_of(wid * APT, APT)
    pltpu.sync_copy(lmx.at[pl.ds(abase, APT)], lmxl)
    pltpu.sync_copy(lmy.at[pl.ds(abase, APT)], lmyl)
    half = jnp.float32(ANCHOR_W / 2)
    for j in range(APT // 16):
        cxv = lmxl[pl.ds(j * 16, 16)]
        cyv = lmyl[pl.ds(j * 16, 16)]
        bx0a[pl.ds(j * 16, 16)] = jnp.clip(((cxv - half) * 0.25).astype(_i32), 0, G - 1)
        bx1a[pl.ds(j * 16, 16)] = jnp.clip(((cxv + half) * 0.25).astype(_i32), 0, G - 1)
        by0a[pl.ds(j * 16, 16)] = jnp.clip(((cyv - half) * 0.25).astype(_i32), 0, G - 1)
        by1a[pl.ds(j * 16, 16)] = jnp.clip(((cyv + half) * 0.25).astype(_i32), 0, G - 1)

    plsc.subcore_barrier()

    pltpu.sync_copy(ovc_sh, ovcv)
    oacc = jnp.zeros((16,), _i32)
    for t in range(NT):
        oacc += ovcv[pl.ds(t * 16, 16)]
    ovtot = jnp.max(oacc)

    hw = jnp.float32(ANCHOR_W / 2)
    denom = hw * hw + jnp.float32(1e-6)

    def extracts(a):
        return (_gv(lmxl, a), _gv(lmyl, a), _gv(bx0a, a), _gv(bx1a, a),
                _gv(by0a, a), _gv(by1a, a))

    def anchor_body(a, carry):
        cx, cy, bx0, bx1, by0, by1 = extracts(a)
        b0 = bx0 * G + by0
        bb0 = pl.multiple_of(b0 * CAP, CAP)
        d0 = pltpu.async_copy(shx.at[pl.ds(bb0, 2 * CAP)], bufx0, sem)
        d1 = pltpu.async_copy(shy.at[pl.ds(bb0, 2 * CAP)], bufy0, sem)
        two_rows = bx1 > bx0

        @pl.when(two_rows)
        def _():
            b1 = bx1 * G + by0
            bb1 = pl.multiple_of(b1 * CAP, CAP)
            d3 = pltpu.async_copy(shx.at[pl.ds(bb1, 2 * CAP)], bufx1, sem)
            d4 = pltpu.async_copy(shy.at[pl.ds(bb1, 2 * CAP)], bufy1, sem)
            d3.wait()
            d4.wait()

        sacc[...] = jnp.zeros((16,), _f32)
        cacc[...] = jnp.zeros((16,), _f32)
        cxv = jnp.full((16,), cx, _f32)
        cyv = jnp.full((16,), cy, _f32)

        d0.wait()
        d1.wait()

        def accum_chunk(xv, yv, lane_ok):
            inbox = (lane_ok
                     & (xv >= cxv - half) & (xv <= cxv + half)
                     & (yv >= cyv - half) & (yv <= cyv + half))
            dx = xv - cxv
            dy = yv - cyv
            r2 = dx * dx + dy * dy
            w = jnp.exp(-r2 / denom)
            sacc[...] += jnp.where(inbox, w, jnp.float32(0.0))
            cacc[...] += jnp.where(inbox, jnp.float32(1.0), jnp.float32(0.0))

        def row_accum(bx, bufx, bufy):
            def by_body(by, carry3):
                b = bx * G + by
                n = jnp.minimum(_gv(tot, b), CAP)
                off = (by - by0) * CAP

                def chunk(k, carry4):
                    o = pl.multiple_of(off + k * 16, 16)
                    lane_ok = (l + k * 16) < n
                    accum_chunk(bufx[pl.ds(o, 16)], bufy[pl.ds(o, 16)],
                                lane_ok)
                    return carry4

                lax.fori_loop(0, (n + 15) // 16, chunk, 0)
                return carry3

            lax.fori_loop(by0, by1 + 1, by_body, 0)

        row_accum(bx0, bufx0, bufy0)

        @pl.when(two_rows)
        def _():
            row_accum(bx1, bufx1, bufy1)

        @pl.when(ovtot > 0)
        def _():
            def t_body(t, carry2):
                ovt = _gv(ovcv, t * 16)

                def ovchunk(k, carry3):
                    ob = pl.multiple_of(OVSTART + t * PPC + k * 16, 16)
                    pltpu.sync_copy(shx.at[pl.ds(ob, 16)],
                                    bufx0.at[pl.ds(0, 16)])
                    pltpu.sync_copy(shy.at[pl.ds(ob, 16)],
                                    bufy0.at[pl.ds(0, 16)])
                    lane_ok = (l + k * 16) < ovt
                    accum_chunk(bufx0[pl.ds(0, 16)], bufy0[pl.ds(0, 16)],
                                lane_ok)
                    return carry3

                lax.fori_loop(0, (ovt + 15) // 16, ovchunk, 0)
                return carry2

            lax.fori_loop(0, NT, t_body, 0)

        s = jnp.sum(sacc[...])
        c = jnp.sum(cacc[...])
        val = jnp.full((16,), s, _f32) / (jnp.full((16,), c, _f32) + 1.0)
        plsc.store_scatter(scl, [jnp.full((16,), a, _i32)], val, mask=l == 0)
        return carry

    lax.fori_loop(0, APT, anchor_body, 0)
    pltpu.sync_copy(scl, scores.at[pl.ds(abase, APT)])


@functools.partial(
    pl.kernel, mesh=_mesh, compiler_params=_sc_params,
    out_type=jax.ShapeDtypeStruct((A,), _f32),
    scratch_types=[pltpu.VMEM((A,), _f32),
                   pltpu.VMEM((A,), _f32),
                   pltpu.VMEM((A,), _f32),
                   pltpu.VMEM((NBPAD,), _i32),
                   pltpu.VMEM((NBPAD,), _i32),
                   pltpu.VMEM((NBPAD,), _i32),
                   pltpu.VMEM((A,), _i32),
                   pltpu.VMEM((APT,), _f32)],
)
def _sc_nms(lmx, lmy, scores, supp_out, acx, acy, asc, histl, off, cur,
            csr, supl):
    cid = lax.axis_index("c")
    sid = lax.axis_index("s")
    wid = cid * NT + sid
    l = lax.iota(_i32, 16)

    pltpu.sync_copy(lmx, acx)
    pltpu.sync_copy(lmy, acy)
    pltpu.sync_copy(scores, asc)

    def zero(c, carry):
        histl[pl.ds(pl.multiple_of(c * 16, 16), 16)] = jnp.zeros((16,), _i32)
        return carry

    lax.fori_loop(0, NBPAD // 16, zero, 0)

    def hchunk(k, carry):
        o = pl.multiple_of(k * 16, 16)
        b = _bin_ids(acx[pl.ds(o, 16)], acy[pl.ds(o, 16)])
        cnt, last = plsc.scan_count(b)
        c0 = plsc.load_gather(histl, [b])
        plsc.store_scatter(histl, [b], c0 + cnt, mask=last)
        return carry

    lax.fori_loop(0, A // 16, hchunk, 0)

    def offchunk(c, carry):
        o = pl.multiple_of(c * 16, 16)
        t16 = histl[pl.ds(o, 16)]
        cs = plsc.cumsum(t16)
        ex = carry + (cs - t16)
        off[pl.ds(o, 16)] = ex
        cur[pl.ds(o, 16)] = ex
        return carry + jnp.max(cs)

    lax.fori_loop(0, NBPAD // 16, offchunk, jnp.int32(0))

    def cchunk(k, carry):
        o = pl.multiple_of(k * 16, 16)
        b = _bin_ids(acx[pl.ds(o, 16)], acy[pl.ds(o, 16)])
        cnt, last = plsc.scan_count(b)
        c0 = plsc.load_gather(cur, [b])
        plsc.store_scatter(cur, [b], c0 + cnt, mask=last)
        plsc.store_scatter(csr, [c0 + cnt - 1], o + l)
        return carry

    lax.fori_loop(0, A // 16, cchunk, 0)

    abase = pl.multiple_of(wid * APT, APT)
    half = jnp.float32(ANCHOR_W / 2)

    def chunk_body(j, carry):
        o = pl.multiple_of(abase + j * 16, 16)
        cxv = acx[pl.ds(o, 16)]
        cyv = acy[pl.ds(o, 16)]
        scv = asc[pl.ds(o, 16)]
        gav = o + l
        bxv = jnp.clip((cxv * 0.25).astype(_i32), 0, G - 1)
        byv = jnp.clip((cyv * 0.25).astype(_i32), 0, G - 1)
        x1i = cxv - half
        x2i = cxv + half
        y1i = cyv - half
        y2i = cyv + half
        area_i = (x2i - x1i) * (y2i - y1i)

        supv = jnp.zeros((16,), jnp.bool_)
        for dx in (-1, 0, 1):
            for dy in (-1, 0, 1):
                bn = (jnp.clip(bxv + dx, 0, G - 1) * G
                      + jnp.clip(byv + dy, 0, G - 1))
                offv = plsc.load_gather(off, [bn])
                cntv = plsc.load_gather(histl, [bn])
                mx = jnp.max(cntv)

                def kbody(k, sup):
                    lane_ok = k < cntv
                    aj = plsc.load_gather(
                        csr, [jnp.minimum(offv + k, A - 1)])
                    cxj = plsc.load_gather(acx, [aj])
                    cyj = plsc.load_gather(acy, [aj])
                    scj = plsc.load_gather(asc, [aj])
                    x1j = cxj - half
                    x2j = cxj + half
                    y1j = cyj - half
                    y2j = cyj + half
                    area_j = (x2j - x1j) * (y2j - y1j)
                    iw = jnp.maximum(
                        jnp.minimum(x2i, x2j) - jnp.maximum(x1i, x1j),
                        jnp.float32(0.0))
                    ih = jnp.maximum(
                        jnp.minimum(y2i, y2j) - jnp.maximum(y1i, y1j),
                        jnp.float32(0.0))
                    inter = iw * ih
                    union = area_i + area_j - inter
                    iou = inter / (union + jnp.float32(1e-9))
                    higher = (scj > scv) | ((scj == scv) & (aj < gav))
                    cond = lane_ok & higher & (iou > jnp.float32(NMS_IOU))
                    return sup | cond

                supv = lax.fori_loop(0, mx, kbody, supv)

        supl[pl.ds(pl.multiple_of(j * 16, 16), 16)] = jnp.where(
            supv, jnp.float32(1.0), jnp.float32(0.0))
        return carry

    lax.fori_loop(0, APT // 16, chunk_body, 0)
    pltpu.sync_copy(supl, supp_out.at[pl.ds(abase, APT)])


def _topk_body(score_c, supp_c, score_r, supp_r, cx_r, cy_r,
               boxes_ref, top_ref):
    neg_inf = jnp.float32(-jnp.inf)
    m_r = jnp.where(supp_r[...] > 0, neg_inf, score_r[...])
    rank = jnp.zeros((1, A), _i32)
    idx_j = jax.lax.broadcasted_iota(_i32, (BA, A), 1)
    for ib in range(A // BA):
        m_i = jnp.where(supp_c[pl.ds(ib * BA, BA), :] > 0, neg_inf,
                        score_c[pl.ds(ib * BA, BA), :])
        idx_i = ib * BA + jax.lax.broadcasted_iota(_i32, (BA, A), 0)
        ahead = (m_i > m_r) | ((m_i == m_r) & (idx_i < idx_j))
        rank = rank + ahead.astype(_i32).sum(axis=0, keepdims=True)

    m_j = m_r
    k = jax.lax.broadcasted_iota(_i32, (TOPK, A), 0)
    eq = rank == k
    zero = jnp.float32(0.0)
    top_ref[...] = jnp.where(eq, m_j, zero).sum(axis=1, keepdims=True)
    bx = jnp.where(eq, cx_r[...], zero).sum(axis=1, keepdims=True)
    by = jnp.where(eq, cy_r[...], zero).sum(axis=1, keepdims=True)
    ones = jnp.ones((TOPK, 1), _f32)
    boxes_ref[...] = jnp.concatenate(
        [bx, by, jnp.zeros((TOPK, 1), _f32),
         ones * jnp.float32(ANCHOR_W), ones * jnp.float32(ANCHOR_L),
         ones * jnp.float32(ANCHOR_H)], axis=1)


def kernel(points, gt_boxes, local_maxima, plot_bounds, training):
    del gt_boxes, plot_bounds, training
    px = points[:, 0].astype(_f32)
    py = points[:, 1].astype(_f32)
    pz = points[:, 2].astype(_f32)
    lmx = local_maxima[:, 0].astype(_f32)
    lmy = local_maxima[:, 1].astype(_f32)

    score_flat = _sc_all(px, py, pz, lmx, lmy)
    supp_flat = _sc_nms(lmx, lmy, score_flat)

    score = score_flat.reshape(A, 1)
    score_r = score_flat.reshape(1, A)
    supp = supp_flat.reshape(A, 1)
    supp_r = supp_flat.reshape(1, A)
    cx_r = lmx.reshape(1, A)
    cy_r = lmy.reshape(1, A)

    fullc = pl.BlockSpec((A, 1), lambda: (0, 0))
    fullr = pl.BlockSpec((1, A), lambda: (0, 0))
    boxes, top = pl.pallas_call(
        _topk_body,
        in_specs=[fullc, fullc, fullr, fullr, fullr, fullr],
        out_specs=[pl.BlockSpec((TOPK, 6), lambda: (0, 0)),
                   pl.BlockSpec((TOPK, 1), lambda: (0, 0))],
        out_shape=[jax.ShapeDtypeStruct((TOPK, 6), _f32),
                   jax.ShapeDtypeStruct((TOPK, 1), _f32)],
    )(score, supp, score_r, supp_r, cx_r, cy_r)

    return boxes, top.reshape(TOPK)
